# baseline (speedup 1.0000x reference)
import jax
import jax.numpy as jnp
from jax.experimental import pallas as pl


def _body(x_ref, st_ref, len_ref):
    x = x_ref[...]
    st_ref[...] = jnp.swapaxes(x, 0, 1)
    rs = jnp.sum(x, axis=2)
    cnt = jnp.sum((rs != 0.0).astype(jnp.int32), axis=0)
    len_ref[...] = cnt[None, :]


def kernel(batch):
    S, B, D = batch.shape
    bB = 128
    states, lengths2d = pl.pallas_call(
        _body,
        grid=(B // bB,),
        in_specs=[pl.BlockSpec((S, bB, D), lambda i: (0, i, 0))],
        out_specs=[
            pl.BlockSpec((bB, S, D), lambda i: (i, 0, 0)),
            pl.BlockSpec((1, bB), lambda i: (0, i)),
        ],
        out_shape=[
            jax.ShapeDtypeStruct((B, S, D), jnp.float32),
            jax.ShapeDtypeStruct((1, B), jnp.int32),
        ],
    )(batch)
    return states, lengths2d.reshape(B)
